# baseline (speedup 1.0000x reference)
import jax
import jax.numpy as jnp
from jax.experimental import pallas as pl
from jax.experimental.pallas import tpu as pltpu

_B, _P, _A, _E, _C = 128, 8, 12, 512, 2
_EC = _E * _C
_AC = _A * _C
_S = _A + 1
_BB = 8
_DELTA = 1e-20


def _body(back_ref, fwd_ref, wsel_ref, bsel_ref, pif_ref, rew_ref, iflow_ref,
          out_ref):
    wsel = wsel_ref[...]
    bsel = bsel_ref[...]

    back = back_ref[...].reshape(_BB * _P * _S, _EC)
    y_back = jnp.dot(back, wsel, preferred_element_type=jnp.float32)
    y_back = y_back.reshape(_BB, _P, _S, _AC) + bsel.reshape(1, 1, 1, _AC)
    sp_back = jax.nn.softplus(y_back)

    s_idx = jax.lax.broadcasted_iota(jnp.int32, (_S, _AC), 0)
    j_idx = jax.lax.broadcasted_iota(jnp.int32, (_S, _AC), 1)
    diag = (j_idx // _C == s_idx - 1).astype(jnp.float32)
    f_in_cols = jnp.sum(sp_back * diag[None, None], axis=2)

    fwd = fwd_ref[...].reshape(_BB * _P, _EC)

    y_fwd = jnp.dot(fwd, wsel, preferred_element_type=jnp.float32)
    y_fwd = y_fwd.reshape(_BB, _P, _AC) + bsel.reshape(1, 1, _AC)
    f_out_cols = jax.nn.softplus(y_fwd)

    par = jax.lax.broadcasted_iota(jnp.int32, (1, 1, _AC), 2) % _C
    even = (par == 0).astype(jnp.float32)
    odd = 1.0 - even
    f_in0 = jnp.sum(f_in_cols * even, axis=-1)
    f_in1 = jnp.sum(f_in_cols * odd, axis=-1)
    f_out0 = jnp.sum(f_out_cols * even, axis=-1)
    f_out1 = jnp.sum(f_out_cols * odd, axis=-1)

    rew = rew_ref[...]
    pif = pif_ref[...]
    iflow = iflow_ref[...]
    f_init = pif * jnp.exp(iflow).reshape(1, 1, _C)

    f_out = jnp.stack([f_out0, f_out1], axis=-1)
    f_in = jnp.stack([f_in0, f_in1], axis=-1)
    logterm = jnp.log(_DELTA + f_out) - jnp.log(_DELTA + f_out + rew)

    x = logterm
    for sh in (1, 2, 4):
        shifted = jnp.concatenate(
            [jnp.zeros((_BB, sh, _C), jnp.float32), x[:, :_P - sh, :]], axis=1)
        x = x + shifted
    p_out = jnp.concatenate(
        [jnp.zeros((_BB, 1, _C), jnp.float32), x[:, :_P - 1, :]], axis=1)

    out_ref[...] = jnp.stack([f_in, f_out, rew, f_init, p_out, rew], axis=-1)


def kernel(forward_edges, backward_edges, path_init_flow, paths_reward, W, b,
           initial_flow):
    fwd = forward_edges.reshape(_B, _P, _S * _EC)
    back = backward_edges.reshape(_B, _P, _S, _EC)

    wt = jnp.transpose(W, (1, 0, 2))
    eye = jnp.eye(_C, dtype=W.dtype)
    wsel = (wt[:, :, :, None] * eye[None, :, None, :]).reshape(_EC, _AC)
    bsel = jnp.transpose(b).reshape(1, _AC)
    iflow = initial_flow.reshape(1, _C)

    grid = (_B // _BB,)
    out = pl.pallas_call(
        _body,
        grid=grid,
        in_specs=[
            pl.BlockSpec((_BB, _P, _S, _EC), lambda i: (i, 0, 0, 0)),
            pl.BlockSpec((_BB, _P, _EC), lambda i: (i, 0, 0)),
            pl.BlockSpec((_EC, _AC), lambda i: (0, 0)),
            pl.BlockSpec((1, _AC), lambda i: (0, 0)),
            pl.BlockSpec((_BB, _P, _C), lambda i: (i, 0, 0)),
            pl.BlockSpec((_BB, _P, _C), lambda i: (i, 0, 0)),
            pl.BlockSpec((1, _C), lambda i: (0, 0)),
        ],
        out_specs=pl.BlockSpec((_BB, _P, _C, 6), lambda i: (i, 0, 0, 0)),
        out_shape=jax.ShapeDtypeStruct((_B, _P, _C, 6), jnp.float32),
        compiler_params=pltpu.CompilerParams(
            dimension_semantics=("arbitrary",)),
    )(back, fwd, wsel, bsel, path_init_flow, paths_reward, iflow)
    return out
